# Initial kernel scaffold; baseline (speedup 1.0000x reference)
import jax
import jax.numpy as jnp
from jax.experimental import pallas as pl


def kernel(edge_feat, edge_index, W1, b1, W2, b2, W3, b3, ln_g, ln_b, Wg, bg):
    raise NotImplementedError("write your pallas kernel here")


# speedup vs baseline: 7.5678x; 7.5678x over previous
import functools

import jax
import jax.numpy as jnp
from jax import lax
from jax.experimental import pallas as pl
from jax.experimental.pallas import tpu as pltpu
from jax.experimental.pallas import tpu_sc as plsc

_N = 100000
_NPAD = 102400
_ROWS_PER_TILE = _NPAD // 16
_ZCHUNK = 400
_NW = 32


def _mlp_ln_tc(edge_feat, W1, b1, W2, b2, W3, b3, ln_g, ln_b):
    E = edge_feat.shape[0]
    BE = 12800
    grid = (E // BE,)

    def body(x_ref, w1_ref, b1_ref, w2_ref, b2_ref, w3_ref, b3_ref,
             g_ref, bb_ref, o_ref):
        x = x_ref[...]
        m = jnp.maximum(jnp.dot(x, w1_ref[...],
                                preferred_element_type=jnp.float32)
                        + b1_ref[...][None, :], 0.0)
        m = jnp.maximum(jnp.dot(m, w2_ref[...],
                                preferred_element_type=jnp.float32)
                        + b2_ref[...][None, :], 0.0)
        m = jnp.dot(m, w3_ref[...],
                    preferred_element_type=jnp.float32) + b3_ref[...][None, :]
        mu = jnp.mean(m, axis=-1, keepdims=True)
        d = m - mu
        var = jnp.mean(d * d, axis=-1, keepdims=True)
        y = d * lax.rsqrt(var + 1e-5)
        o_ref[...] = y * g_ref[...][None, :] + bb_ref[...][None, :]

    wspec = pl.BlockSpec((16, 16), lambda i: (0, 0))
    vspec = pl.BlockSpec((16,), lambda i: (0,))
    return pl.pallas_call(
        body,
        grid=grid,
        in_specs=[pl.BlockSpec((BE, 16), lambda i: (i, 0)),
                  wspec, vspec, wspec, vspec, wspec, vspec, vspec, vspec],
        out_specs=pl.BlockSpec((BE, 16), lambda i: (i, 0)),
        out_shape=jax.ShapeDtypeStruct((E, 16), jnp.float32),
    )(edge_feat, W1, b1, W2, b2, W3, b3, ln_g, ln_b)


def _sc_scatter_msgs(m, dst, src):
    E = m.shape[0]
    C = 800
    per_w = E // _NW
    n_chunks = per_w // C
    mesh = plsc.VectorSubcoreMesh(core_axis_name="c", subcore_axis_name="s")

    @functools.partial(
        pl.kernel,
        out_type=(jax.ShapeDtypeStruct((2, _NPAD, 16), jnp.float32),
                  jax.ShapeDtypeStruct((2, _NPAD), jnp.float32),
                  jax.ShapeDtypeStruct((2, _NPAD), jnp.float32)),
        mesh=mesh,
        compiler_params=pltpu.CompilerParams(use_tc_tiling_on_sc=False),
        scratch_types=(
            pltpu.VMEM_SHARED((_NPAD, 16), jnp.float32),
            pltpu.VMEM_SHARED((_NPAD,), jnp.float32),
            pltpu.VMEM_SHARED((_NPAD,), jnp.float32),
            pltpu.VMEM((C, 16), jnp.float32),
            pltpu.VMEM((C,), jnp.int32),
            pltpu.VMEM((C,), jnp.int32),
            pltpu.VMEM((C,), jnp.float32),
        ),
    )
    def body(m_hbm, dst_hbm, src_hbm, h_out, din_out, dout_out,
             h_sh, din_sh, dout_sh, rows_v, dsti_v, srci_v, ones_v):
        cid = lax.axis_index("c")
        sid = lax.axis_index("s")
        wid = sid * 2 + cid

        zeros16 = jnp.zeros((16,), jnp.float32)
        ones16 = jnp.ones((16,), jnp.float32)

        @pl.loop(0, _ZCHUNK)
        def _(i):
            rows_v[i, :] = zeros16

        @pl.loop(0, _ZCHUNK // 16)
        def _(i):
            ones_v[pl.ds(i * 16, 16)] = zeros16

        tb = sid * _ROWS_PER_TILE

        @pl.loop(0, _ROWS_PER_TILE // _ZCHUNK)
        def _(k):
            o = tb + k * _ZCHUNK
            pltpu.sync_copy(rows_v.at[pl.ds(0, _ZCHUNK), :],
                            h_sh.at[pl.ds(o, _ZCHUNK), :])
            pltpu.sync_copy(ones_v.at[pl.ds(0, _ZCHUNK)],
                            din_sh.at[pl.ds(o, _ZCHUNK)])
            pltpu.sync_copy(ones_v.at[pl.ds(0, _ZCHUNK)],
                            dout_sh.at[pl.ds(o, _ZCHUNK)])

        @pl.loop(0, C // 16)
        def _(i):
            ones_v[pl.ds(i * 16, 16)] = ones16

        plsc.subcore_barrier()

        @pl.loop(0, n_chunks)
        def _(i):
            base = wid * per_w + i * C
            pltpu.sync_copy(m_hbm.at[pl.ds(base, C), :], rows_v)
            pltpu.sync_copy(dst_hbm.at[pl.ds(base, C)], dsti_v)
            pltpu.sync_copy(src_hbm.at[pl.ds(base, C)], srci_v)
            pltpu.sync_copy(rows_v, h_sh.at[dsti_v], add=True)
            pltpu.sync_copy(ones_v, din_sh.at[dsti_v], add=True)
            pltpu.sync_copy(ones_v, dout_sh.at[srci_v], add=True)

        plsc.subcore_barrier()

        pltpu.sync_copy(h_sh.at[pl.ds(tb, _ROWS_PER_TILE), :],
                        h_out.at[cid, pl.ds(tb, _ROWS_PER_TILE), :])
        pltpu.sync_copy(din_sh.at[pl.ds(tb, _ROWS_PER_TILE)],
                        din_out.at[cid, pl.ds(tb, _ROWS_PER_TILE)])
        pltpu.sync_copy(dout_sh.at[pl.ds(tb, _ROWS_PER_TILE)],
                        dout_out.at[cid, pl.ds(tb, _ROWS_PER_TILE)])

    return body(m, dst, src)


def _rsqrt_newton(x):
    i = lax.bitcast_convert_type(x, jnp.int32)
    i = 0x5F3759DF - lax.shift_right_logical(i, 1)
    y = lax.bitcast_convert_type(i, jnp.float32)
    for _ in range(4):
        y = y * (1.5 - 0.5 * x * y * y)
    return y


def _sc_scale(h_p, dout_p, din_p):
    RPW = _NPAD // _NW
    mesh = plsc.VectorSubcoreMesh(core_axis_name="c", subcore_axis_name="s")

    @functools.partial(
        pl.kernel,
        out_type=(jax.ShapeDtypeStruct((_NPAD, 16), jnp.float32),
                  jax.ShapeDtypeStruct((_NPAD,), jnp.float32)),
        mesh=mesh,
        compiler_params=pltpu.CompilerParams(use_tc_tiling_on_sc=False),
        scratch_types=(
            pltpu.VMEM((RPW, 16), jnp.float32),
            pltpu.VMEM((RPW, 16), jnp.float32),
            pltpu.VMEM((RPW,), jnp.float32),
            pltpu.VMEM((RPW,), jnp.float32),
            pltpu.VMEM((RPW,), jnp.float32),
            pltpu.VMEM((RPW,), jnp.float32),
        ),
    )
    def body(hp_hbm, dop_hbm, dip_hbm, feat_out, sin_out,
             h0_v, h1_v, do0_v, do1_v, di0_v, di1_v):
        cid = lax.axis_index("c")
        sid = lax.axis_index("s")
        wid = sid * 2 + cid
        base = wid * RPW

        pltpu.sync_copy(hp_hbm.at[0, pl.ds(base, RPW), :], h0_v)
        pltpu.sync_copy(hp_hbm.at[1, pl.ds(base, RPW), :], h1_v)
        pltpu.sync_copy(dop_hbm.at[0, pl.ds(base, RPW)], do0_v)
        pltpu.sync_copy(dop_hbm.at[1, pl.ds(base, RPW)], do1_v)
        pltpu.sync_copy(dip_hbm.at[0, pl.ds(base, RPW)], di0_v)
        pltpu.sync_copy(dip_hbm.at[1, pl.ds(base, RPW)], di1_v)

        @pl.loop(0, RPW // 16)
        def _(k):
            o = k * 16
            dout = do0_v[pl.ds(o, 16)] + do1_v[pl.ds(o, 16)]
            do0_v[pl.ds(o, 16)] = _rsqrt_newton(jnp.maximum(dout, 1.0))
            din = di0_v[pl.ds(o, 16)] + di1_v[pl.ds(o, 16)]
            di0_v[pl.ds(o, 16)] = _rsqrt_newton(jnp.maximum(din, 1.0))

        @pl.loop(0, RPW // 16)
        def _(k):
            s16 = do0_v[pl.ds(k * 16, 16)]
            for j in range(16):
                r = k * 16 + j
                h0_v[r, :] = (h0_v[r, :] + h1_v[r, :]) * s16[j]

        pltpu.sync_copy(h0_v, feat_out.at[pl.ds(base, RPW), :])
        pltpu.sync_copy(di0_v, sin_out.at[pl.ds(base, RPW)])

    return body(h_p, dout_p, din_p)


def _sc_gather_scatter(feat, src, dst):
    E = src.shape[0]
    C = 1000
    per_w = E // _NW
    n_chunks = per_w // C
    mesh = plsc.VectorSubcoreMesh(core_axis_name="c", subcore_axis_name="s")

    @functools.partial(
        pl.kernel,
        out_type=jax.ShapeDtypeStruct((2, _NPAD, 16), jnp.float32),
        mesh=mesh,
        compiler_params=pltpu.CompilerParams(use_tc_tiling_on_sc=False),
        scratch_types=(
            pltpu.VMEM_SHARED((_NPAD, 16), jnp.float32),
            pltpu.VMEM((C, 16), jnp.float32),
            pltpu.VMEM((C,), jnp.int32),
            pltpu.VMEM((C,), jnp.int32),
            pltpu.SemaphoreType.DMA,
        ),
    )
    def body(feat_hbm, src_hbm, dst_hbm, agg_out,
             agg_sh, rows_v, srci_v, dsti_v, sem):
        cid = lax.axis_index("c")
        sid = lax.axis_index("s")
        wid = sid * 2 + cid

        zeros16 = jnp.zeros((16,), jnp.float32)

        @pl.loop(0, _ZCHUNK)
        def _(i):
            rows_v[i, :] = zeros16

        tb = sid * _ROWS_PER_TILE

        @pl.loop(0, _ROWS_PER_TILE // _ZCHUNK)
        def _(k):
            pltpu.sync_copy(rows_v.at[pl.ds(0, _ZCHUNK), :],
                            agg_sh.at[pl.ds(tb + k * _ZCHUNK, _ZCHUNK), :])

        plsc.subcore_barrier()

        @pl.loop(0, n_chunks)
        def _(i):
            base = wid * per_w + i * C
            pltpu.sync_copy(src_hbm.at[pl.ds(base, C)], srci_v)
            pltpu.sync_copy(dst_hbm.at[pl.ds(base, C)], dsti_v)
            pltpu.async_copy(feat_hbm.at[srci_v], rows_v, sem).wait()
            pltpu.sync_copy(rows_v, agg_sh.at[dsti_v], add=True)

        plsc.subcore_barrier()

        pltpu.sync_copy(agg_sh.at[pl.ds(tb, _ROWS_PER_TILE), :],
                        agg_out.at[cid, pl.ds(tb, _ROWS_PER_TILE), :])

    return body(feat, src, dst)


def _final_tc(agg2, dsc, Wg, bg):
    BN = 2048
    grid = (pl.cdiv(_N, BN),)

    def body(a0_ref, a1_ref, dsc_ref, wg_ref, bg_ref, o_ref):
        a = (a0_ref[...] + a1_ref[...]) * dsc_ref[...][:, None]
        o = jnp.dot(a, wg_ref[...], preferred_element_type=jnp.float32)
        o_ref[...] = jnp.maximum(o + bg_ref[...][None, :], 0.0)

    return pl.pallas_call(
        body,
        grid=grid,
        in_specs=[pl.BlockSpec((BN, 16), lambda i: (i, 0)),
                  pl.BlockSpec((BN, 16), lambda i: (i + _NPAD // BN, 0)),
                  pl.BlockSpec((BN,), lambda i: (i,)),
                  pl.BlockSpec((16, 128), lambda i: (0, 0)),
                  pl.BlockSpec((128,), lambda i: (0,))],
        out_specs=pl.BlockSpec((BN, 128), lambda i: (i, 0)),
        out_shape=jax.ShapeDtypeStruct((_N, 128), jnp.float32),
    )(agg2, agg2, dsc, Wg, bg)


def kernel(edge_feat, edge_index, W1, b1, W2, b2, W3, b3, ln_g, ln_b, Wg, bg):
    src = edge_index[0]
    dst = edge_index[1]
    m = _mlp_ln_tc(edge_feat, W1, b1, W2, b2, W3, b3, ln_g, ln_b)
    h_p, din_p, dout_p = _sc_scatter_msgs(m, dst, src)
    feat, dsc = _sc_scale(h_p, dout_p, din_p)
    agg_p = _sc_gather_scatter(feat, src, dst)
    agg2 = jnp.reshape(agg_p, (2 * _NPAD, 16))
    return _final_tc(agg2, dsc, Wg, bg)


# speedup vs baseline: 12.6956x; 1.6776x over previous
import functools

import jax
import jax.numpy as jnp
from jax import lax
from jax.experimental import pallas as pl
from jax.experimental.pallas import tpu as pltpu
from jax.experimental.pallas import tpu_sc as plsc

_N = 100000
_NPAD = 102400
_ROWS_PER_TILE = _NPAD // 16
_ZCHUNK = 400
_NW = 32


def _mlp_ln_tc(edge_feat, W1, b1, W2, b2, W3, b3, ln_g, ln_b):
    E = edge_feat.shape[0]
    E8 = E // 8
    BEB = 2000
    grid = (E8 // BEB,)

    x8 = jnp.reshape(edge_feat, (E8, 128))
    eye8 = jnp.eye(8, dtype=jnp.float32)
    W1b = jnp.kron(eye8, W1)
    W2b = jnp.kron(eye8, W2)
    W3b = jnp.kron(eye8, W3)
    Gm = jnp.kron(eye8, jnp.full((16, 16), 1.0 / 16.0, jnp.float32))
    b1b = jnp.tile(b1, 8)
    b2b = jnp.tile(b2, 8)
    b3b = jnp.tile(b3, 8)
    gb = jnp.tile(ln_g, 8)
    bb = jnp.tile(ln_b, 8)

    def body(x_ref, w1_ref, w2_ref, w3_ref, gm_ref, b1_ref, b2_ref, b3_ref,
             g_ref, bb_ref, o_ref):
        x = x_ref[...]
        m = jnp.maximum(jnp.dot(x, w1_ref[...],
                                preferred_element_type=jnp.float32)
                        + b1_ref[...][None, :], 0.0)
        m = jnp.maximum(jnp.dot(m, w2_ref[...],
                                preferred_element_type=jnp.float32)
                        + b2_ref[...][None, :], 0.0)
        m = jnp.dot(m, w3_ref[...],
                    preferred_element_type=jnp.float32) + b3_ref[...][None, :]
        gm = gm_ref[...]
        mu = jnp.dot(m, gm, preferred_element_type=jnp.float32)
        d = m - mu
        var = jnp.dot(d * d, gm, preferred_element_type=jnp.float32)
        y = d * lax.rsqrt(var + 1e-5)
        o_ref[...] = y * g_ref[...][None, :] + bb_ref[...][None, :]

    wspec = pl.BlockSpec((128, 128), lambda i: (0, 0))
    vspec = pl.BlockSpec((128,), lambda i: (0,))
    m8 = pl.pallas_call(
        body,
        grid=grid,
        in_specs=[pl.BlockSpec((BEB, 128), lambda i: (i, 0)),
                  wspec, wspec, wspec, wspec,
                  vspec, vspec, vspec, vspec, vspec],
        out_specs=pl.BlockSpec((BEB, 128), lambda i: (i, 0)),
        out_shape=jax.ShapeDtypeStruct((E8, 128), jnp.float32),
    )(x8, W1b, W2b, W3b, Gm, b1b, b2b, b3b, gb, bb)
    return jnp.reshape(m8, (E, 16))


def _sc_scatter_msgs(m, dst, src):
    E = m.shape[0]
    C = 800
    per_w = E // _NW
    n_chunks = per_w // C
    mesh = plsc.VectorSubcoreMesh(core_axis_name="c", subcore_axis_name="s")

    @functools.partial(
        pl.kernel,
        out_type=(jax.ShapeDtypeStruct((2, _NPAD, 16), jnp.float32),
                  jax.ShapeDtypeStruct((2, _NPAD), jnp.float32),
                  jax.ShapeDtypeStruct((2, _NPAD), jnp.float32)),
        mesh=mesh,
        compiler_params=pltpu.CompilerParams(use_tc_tiling_on_sc=False),
        scratch_types=(
            pltpu.VMEM_SHARED((_NPAD, 16), jnp.float32),
            pltpu.VMEM_SHARED((_NPAD,), jnp.float32),
            pltpu.VMEM_SHARED((_NPAD,), jnp.float32),
            pltpu.VMEM((C, 16), jnp.float32),
            pltpu.VMEM((C,), jnp.int32),
            pltpu.VMEM((C,), jnp.int32),
            pltpu.VMEM((C,), jnp.float32),
        ),
    )
    def body(m_hbm, dst_hbm, src_hbm, h_out, din_out, dout_out,
             h_sh, din_sh, dout_sh, rows_v, dsti_v, srci_v, ones_v):
        cid = lax.axis_index("c")
        sid = lax.axis_index("s")
        wid = sid * 2 + cid

        zeros16 = jnp.zeros((16,), jnp.float32)
        ones16 = jnp.ones((16,), jnp.float32)

        @pl.loop(0, _ZCHUNK)
        def _(i):
            rows_v[i, :] = zeros16

        @pl.loop(0, _ZCHUNK // 16)
        def _(i):
            ones_v[pl.ds(i * 16, 16)] = zeros16

        tb = sid * _ROWS_PER_TILE

        @pl.loop(0, _ROWS_PER_TILE // _ZCHUNK)
        def _(k):
            o = tb + k * _ZCHUNK
            pltpu.sync_copy(rows_v.at[pl.ds(0, _ZCHUNK), :],
                            h_sh.at[pl.ds(o, _ZCHUNK), :])
            pltpu.sync_copy(ones_v.at[pl.ds(0, _ZCHUNK)],
                            din_sh.at[pl.ds(o, _ZCHUNK)])
            pltpu.sync_copy(ones_v.at[pl.ds(0, _ZCHUNK)],
                            dout_sh.at[pl.ds(o, _ZCHUNK)])

        @pl.loop(0, C // 16)
        def _(i):
            ones_v[pl.ds(i * 16, 16)] = ones16

        plsc.subcore_barrier()

        @pl.loop(0, n_chunks)
        def _(i):
            base = wid * per_w + i * C
            pltpu.sync_copy(m_hbm.at[pl.ds(base, C), :], rows_v)
            pltpu.sync_copy(dst_hbm.at[pl.ds(base, C)], dsti_v)
            pltpu.sync_copy(src_hbm.at[pl.ds(base, C)], srci_v)
            pltpu.sync_copy(rows_v, h_sh.at[dsti_v], add=True)
            pltpu.sync_copy(ones_v, din_sh.at[dsti_v], add=True)
            pltpu.sync_copy(ones_v, dout_sh.at[srci_v], add=True)

        plsc.subcore_barrier()

        pltpu.sync_copy(h_sh.at[pl.ds(tb, _ROWS_PER_TILE), :],
                        h_out.at[cid, pl.ds(tb, _ROWS_PER_TILE), :])
        pltpu.sync_copy(din_sh.at[pl.ds(tb, _ROWS_PER_TILE)],
                        din_out.at[cid, pl.ds(tb, _ROWS_PER_TILE)])
        pltpu.sync_copy(dout_sh.at[pl.ds(tb, _ROWS_PER_TILE)],
                        dout_out.at[cid, pl.ds(tb, _ROWS_PER_TILE)])

    return body(m, dst, src)


def _rsqrt_newton(x):
    i = lax.bitcast_convert_type(x, jnp.int32)
    i = 0x5F3759DF - lax.shift_right_logical(i, 1)
    y = lax.bitcast_convert_type(i, jnp.float32)
    for _ in range(4):
        y = y * (1.5 - 0.5 * x * y * y)
    return y


def _sc_scale(h_p, dout_p, din_p):
    RPW = _NPAD // _NW
    mesh = plsc.VectorSubcoreMesh(core_axis_name="c", subcore_axis_name="s")

    @functools.partial(
        pl.kernel,
        out_type=(jax.ShapeDtypeStruct((_NPAD, 16), jnp.float32),
                  jax.ShapeDtypeStruct((_NPAD,), jnp.float32)),
        mesh=mesh,
        compiler_params=pltpu.CompilerParams(use_tc_tiling_on_sc=False),
        scratch_types=(
            pltpu.VMEM((RPW, 16), jnp.float32),
            pltpu.VMEM((RPW, 16), jnp.float32),
            pltpu.VMEM((RPW,), jnp.float32),
            pltpu.VMEM((RPW,), jnp.float32),
            pltpu.VMEM((RPW,), jnp.float32),
            pltpu.VMEM((RPW,), jnp.float32),
        ),
    )
    def body(hp_hbm, dop_hbm, dip_hbm, feat_out, sin_out,
             h0_v, h1_v, do0_v, do1_v, di0_v, di1_v):
        cid = lax.axis_index("c")
        sid = lax.axis_index("s")
        wid = sid * 2 + cid
        base = wid * RPW

        pltpu.sync_copy(hp_hbm.at[0, pl.ds(base, RPW), :], h0_v)
        pltpu.sync_copy(hp_hbm.at[1, pl.ds(base, RPW), :], h1_v)
        pltpu.sync_copy(dop_hbm.at[0, pl.ds(base, RPW)], do0_v)
        pltpu.sync_copy(dop_hbm.at[1, pl.ds(base, RPW)], do1_v)
        pltpu.sync_copy(dip_hbm.at[0, pl.ds(base, RPW)], di0_v)
        pltpu.sync_copy(dip_hbm.at[1, pl.ds(base, RPW)], di1_v)

        @pl.loop(0, RPW // 16)
        def _(k):
            o = k * 16
            dout = do0_v[pl.ds(o, 16)] + do1_v[pl.ds(o, 16)]
            do0_v[pl.ds(o, 16)] = _rsqrt_newton(jnp.maximum(dout, 1.0))
            din = di0_v[pl.ds(o, 16)] + di1_v[pl.ds(o, 16)]
            di0_v[pl.ds(o, 16)] = _rsqrt_newton(jnp.maximum(din, 1.0))

        @pl.loop(0, RPW // 16)
        def _(k):
            s16 = do0_v[pl.ds(k * 16, 16)]
            for j in range(16):
                r = k * 16 + j
                h0_v[r, :] = (h0_v[r, :] + h1_v[r, :]) * s16[j]

        pltpu.sync_copy(h0_v, feat_out.at[pl.ds(base, RPW), :])
        pltpu.sync_copy(di0_v, sin_out.at[pl.ds(base, RPW)])

    return body(h_p, dout_p, din_p)


def _sc_gather_scatter(feat, src, dst):
    E = src.shape[0]
    C = 1000
    per_w = E // _NW
    n_chunks = per_w // C
    mesh = plsc.VectorSubcoreMesh(core_axis_name="c", subcore_axis_name="s")

    @functools.partial(
        pl.kernel,
        out_type=jax.ShapeDtypeStruct((2, _NPAD, 16), jnp.float32),
        mesh=mesh,
        compiler_params=pltpu.CompilerParams(use_tc_tiling_on_sc=False),
        scratch_types=(
            pltpu.VMEM_SHARED((_NPAD, 16), jnp.float32),
            pltpu.VMEM((C, 16), jnp.float32),
            pltpu.VMEM((C,), jnp.int32),
            pltpu.VMEM((C,), jnp.int32),
            pltpu.SemaphoreType.DMA,
        ),
    )
    def body(feat_hbm, src_hbm, dst_hbm, agg_out,
             agg_sh, rows_v, srci_v, dsti_v, sem):
        cid = lax.axis_index("c")
        sid = lax.axis_index("s")
        wid = sid * 2 + cid

        zeros16 = jnp.zeros((16,), jnp.float32)

        @pl.loop(0, _ZCHUNK)
        def _(i):
            rows_v[i, :] = zeros16

        tb = sid * _ROWS_PER_TILE

        @pl.loop(0, _ROWS_PER_TILE // _ZCHUNK)
        def _(k):
            pltpu.sync_copy(rows_v.at[pl.ds(0, _ZCHUNK), :],
                            agg_sh.at[pl.ds(tb + k * _ZCHUNK, _ZCHUNK), :])

        plsc.subcore_barrier()

        @pl.loop(0, n_chunks)
        def _(i):
            base = wid * per_w + i * C
            pltpu.sync_copy(src_hbm.at[pl.ds(base, C)], srci_v)
            pltpu.sync_copy(dst_hbm.at[pl.ds(base, C)], dsti_v)
            pltpu.async_copy(feat_hbm.at[srci_v], rows_v, sem).wait()
            pltpu.sync_copy(rows_v, agg_sh.at[dsti_v], add=True)

        plsc.subcore_barrier()

        pltpu.sync_copy(agg_sh.at[pl.ds(tb, _ROWS_PER_TILE), :],
                        agg_out.at[cid, pl.ds(tb, _ROWS_PER_TILE), :])

    return body(feat, src, dst)


def _final_tc(agg2, dsc, Wg, bg):
    BN = 2048
    grid = (pl.cdiv(_N, BN),)

    def body(a0_ref, a1_ref, dsc_ref, wg_ref, bg_ref, o_ref):
        a = (a0_ref[...] + a1_ref[...]) * dsc_ref[...][:, None]
        o = jnp.dot(a, wg_ref[...], preferred_element_type=jnp.float32)
        o_ref[...] = jnp.maximum(o + bg_ref[...][None, :], 0.0)

    return pl.pallas_call(
        body,
        grid=grid,
        in_specs=[pl.BlockSpec((BN, 16), lambda i: (i, 0)),
                  pl.BlockSpec((BN, 16), lambda i: (i + _NPAD // BN, 0)),
                  pl.BlockSpec((BN,), lambda i: (i,)),
                  pl.BlockSpec((16, 128), lambda i: (0, 0)),
                  pl.BlockSpec((128,), lambda i: (0,))],
        out_specs=pl.BlockSpec((BN, 128), lambda i: (i, 0)),
        out_shape=jax.ShapeDtypeStruct((_N, 128), jnp.float32),
    )(agg2, agg2, dsc, Wg, bg)


def kernel(edge_feat, edge_index, W1, b1, W2, b2, W3, b3, ln_g, ln_b, Wg, bg):
    src = edge_index[0]
    dst = edge_index[1]
    m = _mlp_ln_tc(edge_feat, W1, b1, W2, b2, W3, b3, ln_g, ln_b)
    h_p, din_p, dout_p = _sc_scatter_msgs(m, dst, src)
    feat, dsc = _sc_scale(h_p, dout_p, din_p)
    agg_p = _sc_gather_scatter(feat, src, dst)
    agg2 = jnp.reshape(agg_p, (2 * _NPAD, 16))
    return _final_tc(agg2, dsc, Wg, bg)
